# Initial kernel scaffold; baseline (speedup 1.0000x reference)
import jax
import jax.numpy as jnp
from jax.experimental import pallas as pl


def kernel(x, edge_index, W1, b1, Wm, bm, W2, b2, gamma, beta, alpha):
    raise NotImplementedError("write your pallas kernel here")


# speedup vs baseline: 4.1201x; 4.1201x over previous
import functools

import jax
import jax.numpy as jnp
from jax import lax
from jax.experimental import pallas as pl
from jax.experimental.pallas import tpu as pltpu
from jax.experimental.pallas import tpu_sc as plsc

N = 10000
E = 160000
NP = 10240
EP = 163840
NCORE = 2
NSUB = 16
RPS = NP // NSUB
EPS_AGG = EP // NSUB
EPS_DEG = EP // (NCORE * NSUB)
BLK = 128
F32 = jnp.float32


def _sc_mesh():
    return plsc.VectorSubcoreMesh(core_axis_name="c", subcore_axis_name="s")


def _sc_deg(dst_p, ones_c, zeros_c):
    @functools.partial(
        pl.kernel,
        out_type=jax.ShapeDtypeStruct((NCORE, NP, 16), F32),
        mesh=_sc_mesh(),
        scratch_types=[
            pltpu.VMEM((BLK,), jnp.int32),
            pltpu.VMEM((BLK, 16), F32),
            pltpu.VMEM_SHARED((NP, 16), F32),
        ],
    )
    def deg_kernel(dst_hbm, ones_hbm, zeros_hbm, out_hbm, idx_v, ones_v, acc):
        c = lax.axis_index("c")
        s = lax.axis_index("s")
        pltpu.sync_copy(zeros_hbm.at[pl.ds(s * RPS, RPS)],
                        acc.at[pl.ds(s * RPS, RPS)])
        pltpu.sync_copy(ones_hbm, ones_v)
        plsc.subcore_barrier()
        base0 = (c * NSUB + s) * EPS_DEG

        @pl.loop(0, EPS_DEG // BLK)
        def _(i):
            pltpu.sync_copy(dst_hbm.at[pl.ds(base0 + i * BLK, BLK)], idx_v)
            pltpu.sync_copy(ones_v, acc.at[idx_v], add=True)

        plsc.subcore_barrier()
        pltpu.sync_copy(acc.at[pl.ds(s * RPS, RPS)],
                        out_hbm.at[c, pl.ds(s * RPS, RPS)])

    return deg_kernel(dst_p, ones_c, zeros_c)


def _make_sc_agg(c_chunks):
    cpc = c_chunks // NCORE

    @functools.partial(
        pl.kernel,
        out_type=jax.ShapeDtypeStruct((c_chunks * NP, 128), F32),
        mesh=_sc_mesh(),
        scratch_types=[
            pltpu.VMEM((BLK,), jnp.int32),
            pltpu.VMEM((BLK,), jnp.int32),
            pltpu.VMEM((BLK, 128), F32),
            pltpu.VMEM_SHARED((NP, 128), F32),
        ],
    )
    def agg_kernel(hs_hbm, srcs_hbm, dst_hbm, out_hbm,
                   idx_s, idx_d, rows_v, acc):
        c = lax.axis_index("c")
        s = lax.axis_index("s")
        for k in range(cpc):
            chunk = k * NCORE + c
            rowoff = chunk * NP
            pltpu.sync_copy(hs_hbm.at[pl.ds(rowoff + s * RPS, RPS)],
                            acc.at[pl.ds(s * RPS, RPS)])
            plsc.subcore_barrier()

            @pl.loop(0, EPS_AGG // BLK)
            def _(i):
                base = s * EPS_AGG + i * BLK
                pltpu.sync_copy(srcs_hbm.at[chunk, pl.ds(base, BLK)], idx_s)
                pltpu.sync_copy(dst_hbm.at[pl.ds(base, BLK)], idx_d)
                pltpu.sync_copy(hs_hbm.at[idx_s], rows_v)
                pltpu.sync_copy(rows_v, acc.at[idx_d], add=True)

            plsc.subcore_barrier()
            pltpu.sync_copy(acc.at[pl.ds(s * RPS, RPS)],
                            out_hbm.at[pl.ds(rowoff + s * RPS, RPS)])

    return agg_kernel


_sc_agg4 = _make_sc_agg(4)
_sc_agg2 = _make_sc_agg(2)


def _tc_prep(deg_part, x_pad):
    def body(dp_ref, x_ref, dinv_ref, xs_ref):
        d = dp_ref[0, :, 0:1] + dp_ref[1, :, 0:1] + 1.0
        r = lax.rsqrt(d)
        row = lax.broadcasted_iota(jnp.int32, (NP, 1), 0)
        dinv = jnp.where(row < N, r, 0.0)
        dinv_ref[...] = dinv
        xs_ref[...] = x_ref[...] * dinv

    return pl.pallas_call(
        body,
        out_shape=(jax.ShapeDtypeStruct((NP, 1), F32),
                   jax.ShapeDtypeStruct(x_pad.shape, F32)),
    )(deg_part, x_pad)


MM_RB = 2560


def _tc_mm(z, w, c_out):
    k_dim = z.shape[1]
    nrb = NP // MM_RB

    def body(z_ref, w_ref, out_ref):
        out_ref[...] = lax.dot_general(
            z_ref[...], w_ref[...], (((1,), (0,)), ((), ())),
            precision=lax.Precision.HIGHEST, preferred_element_type=F32)

    return pl.pallas_call(
        body,
        grid=(nrb, c_out),
        in_specs=[
            pl.BlockSpec((MM_RB, k_dim), lambda rb, co: (rb, 0)),
            pl.BlockSpec((k_dim, 128), lambda rb, co: (0, co)),
        ],
        out_specs=pl.BlockSpec(
            (MM_RB, 128), lambda rb, co: (co * nrb + rb, 0)),
        out_shape=jax.ShapeDtypeStruct((c_out * NP, 128), F32),
    )(z, w)


def _tc_bn_prelu(agg, dinv, b, gamma, beta, alpha, c_in):
    def body(a_ref, dinv_ref, b_ref, g_ref, be_ref, al_ref, out_ref):
        dinv = dinv_ref[...][:N]
        z = a_ref[0][:N] * dinv + b_ref[0]
        mu = jnp.mean(z, axis=0, keepdims=True)
        var = jnp.mean(z * z, axis=0, keepdims=True) - mu * mu
        zn = (z - mu) * lax.rsqrt(var + 1e-5) * g_ref[0] + be_ref[0]
        al = al_ref[0, 0]
        out_ref[:N, :] = jnp.where(zn > 0, zn, al * zn) * dinv
        out_ref[N:, :] = jnp.zeros((NP - N, 128), F32)

    return pl.pallas_call(
        body,
        grid=(c_in,),
        in_specs=[
            pl.BlockSpec((1, NP, 128), lambda ci: (ci, 0, 0)),
            pl.BlockSpec((NP, 1), lambda ci: (0, 0)),
            pl.BlockSpec((1, 1, 128), lambda ci: (ci, 0, 0)),
            pl.BlockSpec((1, 1, 128), lambda ci: (ci, 0, 0)),
            pl.BlockSpec((1, 1, 128), lambda ci: (ci, 0, 0)),
            pl.BlockSpec((1, 1), lambda ci: (0, 0)),
        ],
        out_specs=pl.BlockSpec((NP, 128), lambda ci: (0, ci)),
        out_shape=jax.ShapeDtypeStruct((NP, c_in * 128), F32),
    )(agg, dinv, b, gamma, beta, alpha)


def _tc_final(agg, dinv, b, c_in):
    def body(a_ref, dinv_ref, b_ref, out_ref):
        out_ref[...] = a_ref[0][:N] * dinv_ref[...][:N] + b_ref[0]

    return pl.pallas_call(
        body,
        grid=(c_in,),
        in_specs=[
            pl.BlockSpec((1, NP, 128), lambda ci: (ci, 0, 0)),
            pl.BlockSpec((NP, 1), lambda ci: (0, 0)),
            pl.BlockSpec((1, 1, 128), lambda ci: (ci, 0, 0)),
        ],
        out_specs=pl.BlockSpec((N, 128), lambda ci: (0, ci)),
        out_shape=jax.ShapeDtypeStruct((N, c_in * 128), F32),
    )(agg, dinv, b)


def kernel(x, edge_index, W1, b1, Wm, bm, W2, b2, gamma, beta, alpha):
    src = edge_index[0].astype(jnp.int32)
    dst = edge_index[1].astype(jnp.int32)
    pad = jnp.full((EP - E,), N, jnp.int32)
    src_p = jnp.concatenate([src, pad])
    dst_p = jnp.concatenate([dst, pad])
    shift4 = (jnp.arange(4, dtype=jnp.int32) * NP)[:, None]
    srcs4 = src_p[None, :] + shift4
    srcs2 = src_p[None, :] + shift4[:2]

    ones_c = jnp.ones((BLK, 16), F32)
    zeros_c = jnp.zeros((NP, 16), F32)

    b1r = b1.reshape(4, 1, 128)
    bmr = bm.reshape(4, 1, 128)
    b2r = b2.reshape(2, 1, 128)
    gr = gamma.reshape(4, 1, 128)
    ber = beta.reshape(4, 1, 128)
    al = alpha.reshape(1, 1)

    x_pad = jnp.pad(x, ((0, NP - N), (0, 0)))

    deg_part = _sc_deg(dst_p, ones_c, zeros_c)
    dinv, xs = _tc_prep(deg_part, x_pad)

    hs1 = _tc_mm(xs, W1, 4)
    agg1 = _sc_agg4(hs1, srcs4, dst_p)
    zs1 = _tc_bn_prelu(agg1.reshape(4, NP, 128), dinv, b1r, gr, ber, al, 4)

    hs2 = _tc_mm(zs1, Wm, 4)
    agg2 = _sc_agg4(hs2, srcs4, dst_p)
    zs2 = _tc_bn_prelu(agg2.reshape(4, NP, 128), dinv, bmr, gr, ber, al, 4)

    hs3 = _tc_mm(zs2, W2, 2)
    agg3 = _sc_agg2(hs3, srcs2, dst_p)
    return _tc_final(agg3.reshape(2, NP, 128), dinv, b2r, 2)


# speedup vs baseline: 4.8856x; 1.1858x over previous
import functools

import jax
import jax.numpy as jnp
from jax import lax
from jax.experimental import pallas as pl
from jax.experimental.pallas import tpu as pltpu
from jax.experimental.pallas import tpu_sc as plsc

N = 10000
E = 160000
NP = 10240
EP = 163840
NCORE = 2
NSUB = 16
RPS = NP // NSUB
EPS_AGG = EP // NSUB
EPS_DEG = EP // (NCORE * NSUB)
BLK = 128
F32 = jnp.float32


def _sc_mesh():
    return plsc.VectorSubcoreMesh(core_axis_name="c", subcore_axis_name="s")


def _sc_deg(dst_p, ones_c, zeros_c):
    nb = EPS_DEG // BLK

    @functools.partial(
        pl.kernel,
        out_type=jax.ShapeDtypeStruct((NCORE, NP, 16), F32),
        mesh=_sc_mesh(),
        scratch_types=[
            pltpu.VMEM((BLK,), jnp.int32),
            pltpu.VMEM((BLK, 16), F32),
            pltpu.VMEM_SHARED((NP, 16), F32),
        ],
    )
    def deg_kernel(dst_hbm, ones_hbm, zeros_hbm, out_hbm, idx_v, ones_v, acc):
        c = lax.axis_index("c")
        s = lax.axis_index("s")
        pltpu.sync_copy(zeros_hbm.at[pl.ds(s * RPS, RPS)],
                        acc.at[pl.ds(s * RPS, RPS)])
        pltpu.sync_copy(ones_hbm, ones_v)
        plsc.subcore_barrier()
        base0 = (c * NSUB + s) * EPS_DEG

        @pl.loop(0, nb)
        def _(i):
            pltpu.sync_copy(dst_hbm.at[pl.ds(base0 + i * BLK, BLK)], idx_v)
            pltpu.sync_copy(ones_v, acc.at[idx_v], add=True)

        plsc.subcore_barrier()
        pltpu.sync_copy(acc.at[pl.ds(s * RPS, RPS)],
                        out_hbm.at[c, pl.ds(s * RPS, RPS)])

    return deg_kernel(dst_p, ones_c, zeros_c)


def _make_sc_agg(c_chunks):
    cpc = c_chunks // NCORE
    nb = EPS_AGG // BLK
    nj = nb // 2

    @functools.partial(
        pl.kernel,
        out_type=jax.ShapeDtypeStruct((c_chunks * NP, 128), F32),
        mesh=_sc_mesh(),
        scratch_types=[pltpu.VMEM_SHARED((NP, 128), F32)]
          + [pltpu.VMEM((BLK,), jnp.int32)] * 4
          + [pltpu.VMEM((BLK, 128), F32)] * 2
          + [pltpu.SemaphoreType.DMA] * 6,
    )
    def agg_kernel(hs_hbm, srcs_hbm, dst_hbm, out_hbm, acc,
                   is0, is1, id0, id1, r0, r1,
                   gi0, gi1, gd0, gd1, gg0, gg1):
        ids = [is0, is1]
        idd = [id0, id1]
        rows = [r0, r1]
        isem = [gi0, gi1]
        dsem = [gd0, gd1]
        gsem = [gg0, gg1]
        c = lax.axis_index("c")
        s = lax.axis_index("s")
        ebase = s * EPS_AGG

        for k in range(cpc):
            chunk = k * NCORE + c
            rowoff = chunk * NP

            def src_idx(b, p):
                return pltpu.make_async_copy(
                    srcs_hbm.at[chunk, pl.ds(ebase + b * BLK, BLK)],
                    ids[p], isem[p])

            def dst_idx(b, p):
                return pltpu.make_async_copy(
                    dst_hbm.at[pl.ds(ebase + b * BLK, BLK)], idd[p], dsem[p])

            def gth(p):
                return pltpu.make_async_copy(
                    hs_hbm.at[ids[p]], rows[p], gsem[p])

            pltpu.sync_copy(hs_hbm.at[pl.ds(rowoff + s * RPS, RPS)],
                            acc.at[pl.ds(s * RPS, RPS)])
            plsc.subcore_barrier()

            pltpu.sync_copy(srcs_hbm.at[chunk, pl.ds(ebase, BLK)], ids[0])
            gth(0).start()

            @pl.loop(0, nj - 1)
            def _(j):
                b0 = 2 * j
                for p in range(2):
                    q, b = 1 - p, b0 + p
                    gth(p).wait()
                    pltpu.sync_copy(
                        srcs_hbm.at[chunk, pl.ds(ebase + (b + 1) * BLK, BLK)],
                        ids[q])
                    gth(q).start()
                    pltpu.sync_copy(
                        dst_hbm.at[pl.ds(ebase + b * BLK, BLK)], idd[p])
                    pltpu.sync_copy(rows[p], acc.at[idd[p]], add=True)

            gth(0).wait()
            pltpu.sync_copy(
                srcs_hbm.at[chunk, pl.ds(ebase + (nb - 1) * BLK, BLK)], ids[1])
            gth(1).start()
            pltpu.sync_copy(
                dst_hbm.at[pl.ds(ebase + (nb - 2) * BLK, BLK)], idd[0])
            pltpu.sync_copy(rows[0], acc.at[idd[0]], add=True)
            gth(1).wait()
            pltpu.sync_copy(
                dst_hbm.at[pl.ds(ebase + (nb - 1) * BLK, BLK)], idd[1])
            pltpu.sync_copy(rows[1], acc.at[idd[1]], add=True)

            plsc.subcore_barrier()
            pltpu.sync_copy(acc.at[pl.ds(s * RPS, RPS)],
                            out_hbm.at[pl.ds(rowoff + s * RPS, RPS)])

    return agg_kernel


_sc_agg4 = _make_sc_agg(4)
_sc_agg2 = _make_sc_agg(2)


def _tc_prep(deg_part, x_pad):
    def body(dp_ref, x_ref, dinv_ref, xs_ref):
        d = dp_ref[0, :, 0:1] + dp_ref[1, :, 0:1] + 1.0
        r = lax.rsqrt(d)
        row = lax.broadcasted_iota(jnp.int32, (NP, 1), 0)
        dinv = jnp.where(row < N, r, 0.0)
        dinv_ref[...] = dinv
        xs_ref[...] = x_ref[...] * dinv

    return pl.pallas_call(
        body,
        out_shape=(jax.ShapeDtypeStruct((NP, 1), F32),
                   jax.ShapeDtypeStruct(x_pad.shape, F32)),
    )(deg_part, x_pad)


MM_RB = 2560


def _tc_mm(z, w, c_out):
    k_dim = z.shape[1]
    nrb = NP // MM_RB

    def body(z_ref, w_ref, out_ref):
        out_ref[...] = lax.dot_general(
            z_ref[...], w_ref[...], (((1,), (0,)), ((), ())),
            precision=lax.Precision.HIGHEST, preferred_element_type=F32)

    return pl.pallas_call(
        body,
        grid=(nrb, c_out),
        in_specs=[
            pl.BlockSpec((MM_RB, k_dim), lambda rb, co: (rb, 0)),
            pl.BlockSpec((k_dim, 128), lambda rb, co: (0, co)),
        ],
        out_specs=pl.BlockSpec(
            (MM_RB, 128), lambda rb, co: (co * nrb + rb, 0)),
        out_shape=jax.ShapeDtypeStruct((c_out * NP, 128), F32),
    )(z, w)


def _tc_bn_prelu(agg, dinv, b, gamma, beta, alpha, c_in):
    def body(a_ref, dinv_ref, b_ref, g_ref, be_ref, al_ref, out_ref):
        dinv = dinv_ref[...][:N]
        z = a_ref[0][:N] * dinv + b_ref[0]
        mu = jnp.mean(z, axis=0, keepdims=True)
        var = jnp.mean(z * z, axis=0, keepdims=True) - mu * mu
        zn = (z - mu) * lax.rsqrt(var + 1e-5) * g_ref[0] + be_ref[0]
        al = al_ref[0, 0]
        out_ref[:N, :] = jnp.where(zn > 0, zn, al * zn) * dinv
        out_ref[N:, :] = jnp.zeros((NP - N, 128), F32)

    return pl.pallas_call(
        body,
        grid=(c_in,),
        in_specs=[
            pl.BlockSpec((1, NP, 128), lambda ci: (ci, 0, 0)),
            pl.BlockSpec((NP, 1), lambda ci: (0, 0)),
            pl.BlockSpec((1, 1, 128), lambda ci: (ci, 0, 0)),
            pl.BlockSpec((1, 1, 128), lambda ci: (ci, 0, 0)),
            pl.BlockSpec((1, 1, 128), lambda ci: (ci, 0, 0)),
            pl.BlockSpec((1, 1), lambda ci: (0, 0)),
        ],
        out_specs=pl.BlockSpec((NP, 128), lambda ci: (0, ci)),
        out_shape=jax.ShapeDtypeStruct((NP, c_in * 128), F32),
    )(agg, dinv, b, gamma, beta, alpha)


def _tc_final(agg, dinv, b, c_in):
    def body(a_ref, dinv_ref, b_ref, out_ref):
        out_ref[...] = a_ref[0][:N] * dinv_ref[...][:N] + b_ref[0]

    return pl.pallas_call(
        body,
        grid=(c_in,),
        in_specs=[
            pl.BlockSpec((1, NP, 128), lambda ci: (ci, 0, 0)),
            pl.BlockSpec((NP, 1), lambda ci: (0, 0)),
            pl.BlockSpec((1, 1, 128), lambda ci: (ci, 0, 0)),
        ],
        out_specs=pl.BlockSpec((N, 128), lambda ci: (0, ci)),
        out_shape=jax.ShapeDtypeStruct((N, c_in * 128), F32),
    )(agg, dinv, b)


def kernel(x, edge_index, W1, b1, Wm, bm, W2, b2, gamma, beta, alpha):
    src = edge_index[0].astype(jnp.int32)
    dst = edge_index[1].astype(jnp.int32)
    pad = jnp.full((EP - E,), N, jnp.int32)
    src_p = jnp.concatenate([src, pad])
    dst_p = jnp.concatenate([dst, pad])
    shift4 = (jnp.arange(4, dtype=jnp.int32) * NP)[:, None]
    srcs4 = src_p[None, :] + shift4
    srcs2 = src_p[None, :] + shift4[:2]

    ones_c = jnp.ones((BLK, 16), F32)
    zeros_c = jnp.zeros((NP, 16), F32)

    b1r = b1.reshape(4, 1, 128)
    bmr = bm.reshape(4, 1, 128)
    b2r = b2.reshape(2, 1, 128)
    gr = gamma.reshape(4, 1, 128)
    ber = beta.reshape(4, 1, 128)
    al = alpha.reshape(1, 1)

    x_pad = jnp.pad(x, ((0, NP - N), (0, 0)))

    deg_part = _sc_deg(dst_p, ones_c, zeros_c)
    dinv, xs = _tc_prep(deg_part, x_pad)

    hs1 = _tc_mm(xs, W1, 4)
    agg1 = _sc_agg4(hs1, srcs4, dst_p)
    zs1 = _tc_bn_prelu(agg1.reshape(4, NP, 128), dinv, b1r, gr, ber, al, 4)

    hs2 = _tc_mm(zs1, Wm, 4)
    agg2 = _sc_agg4(hs2, srcs4, dst_p)
    zs2 = _tc_bn_prelu(agg2.reshape(4, NP, 128), dinv, bmr, gr, ber, al, 4)

    hs3 = _tc_mm(zs2, W2, 2)
    agg3 = _sc_agg2(hs3, srcs2, dst_p)
    return _tc_final(agg3.reshape(2, NP, 128), dinv, b2r, 2)
